# baseline (speedup 1.0000x reference)
import functools

import jax
import jax.numpy as jnp
from jax import lax
from jax.experimental import pallas as pl
from jax.experimental.pallas import tpu as pltpu
from jax.experimental.pallas import tpu_sc as plsc

N_ATOMS = 10000
MAX_NB = 32
N_BONDS = 320000
ATOM_FDIM = 128
BOND_FDIM = 144
HID = 300

L = 16
H = 384
NV = H // L
NC, NS = 2, 16
NW = NC * NS
NA = 10240
APW = NA // NW
BPW = N_BONDS // NW
CA = 2
CB = 40

_SC_MESH = dict(core_axis_name="c", subcore_axis_name="s")


def _mm_relu_body(x_ref, w_ref, o_ref):
    o_ref[...] = jnp.maximum(
        jnp.dot(x_ref[...], w_ref[...], preferred_element_type=jnp.float32), 0.0)


def mm_relu(x, w, bm):
    m, k = x.shape
    n = w.shape[1]
    return pl.pallas_call(
        _mm_relu_body,
        grid=(m // bm,),
        in_specs=[pl.BlockSpec((bm, k), lambda i: (i, 0)),
                  pl.BlockSpec((k, n), lambda i: (0, 0))],
        out_specs=pl.BlockSpec((bm, n), lambda i: (i, 0)),
        out_shape=jax.ShapeDtypeStruct((m, n), jnp.float32),
    )(x, w)


def _upd_body(ma_ref, agg_ref, man_ref):
    man_ref[...] = ma_ref[...] + agg_ref[...]


def upd_add(msg_atom, agg, bm):
    m = msg_atom.shape[0]
    return pl.pallas_call(
        _upd_body,
        grid=(m // bm,),
        in_specs=[pl.BlockSpec((bm, H), lambda i: (i, 0)),
                  pl.BlockSpec((bm, H), lambda i: (i, 0))],
        out_specs=pl.BlockSpec((bm, H), lambda i: (i, 0)),
        out_shape=jax.ShapeDtypeStruct((m, H), jnp.float32),
    )(msg_atom, agg)


def _comb_body(pre_ref, w_ref, ib_ref, o_ref):
    o_ref[...] = jnp.maximum(
        ib_ref[...]
        + jnp.dot(pre_ref[...], w_ref[...], preferred_element_type=jnp.float32),
        0.0)


def comb_mm(pre, w, input_bond, bm):
    m = pre.shape[0]
    return pl.pallas_call(
        _comb_body,
        grid=(m // bm,),
        in_specs=[pl.BlockSpec((bm, H), lambda i: (i, 0)),
                  pl.BlockSpec((H, H), lambda i: (0, 0)),
                  pl.BlockSpec((bm, H), lambda i: (i, 0))],
        out_specs=pl.BlockSpec((bm, H), lambda i: (i, 0)),
        out_shape=jax.ShapeDtypeStruct((m, H), jnp.float32),
    )(pre, w, input_bond)


def _final_body(xa_ref, xb_ref, xc_ref, wa_ref, wb_ref, wc_ref, o_ref):
    acc = jnp.dot(xa_ref[...], wa_ref[...], preferred_element_type=jnp.float32)
    acc += jnp.dot(xb_ref[...], wb_ref[...], preferred_element_type=jnp.float32)
    acc += jnp.dot(xc_ref[...], wc_ref[...], preferred_element_type=jnp.float32)
    o_ref[...] = jnp.maximum(acc, 0.0)


def final_mm(xa, xb, xc, wa, wb, wc, bm):
    m = xa.shape[0]
    xspec = pl.BlockSpec((bm, H), lambda i: (i, 0))
    wspec = pl.BlockSpec((H, H), lambda i: (0, 0))
    return pl.pallas_call(
        _final_body,
        grid=(m // bm,),
        in_specs=[xspec, xspec, xspec, wspec, wspec, wspec],
        out_specs=pl.BlockSpec((bm, H), lambda i: (i, 0)),
        out_shape=jax.ShapeDtypeStruct((m, H), jnp.float32),
    )(xa, xb, xc, wa, wb, wc)


def gather_reduce(msg_bond, a2b_flat):
    @functools.partial(
        pl.kernel,
        mesh=plsc.VectorSubcoreMesh(**_SC_MESH),
        out_type=jax.ShapeDtypeStruct((NA, H), jnp.float32),
        scratch_types=[
            pltpu.VMEM((CA * MAX_NB,), jnp.int32),
            pltpu.VMEM((CA * MAX_NB, H), jnp.float32),
            pltpu.VMEM((CA, H), jnp.float32),
            pltpu.SemaphoreType.DMA,
        ],
    )
    def k(msgb_hbm, a2b_hbm, agg_hbm, idx_v, rows_v, agg_v, sem):
        wid = lax.axis_index("s") * NC + lax.axis_index("c")

        def body(i, carry):
            a0 = wid * APW + i * CA
            ib = pl.multiple_of(a0 * MAX_NB, CA * MAX_NB)
            pltpu.sync_copy(a2b_hbm.at[pl.ds(ib, CA * MAX_NB)], idx_v)
            pltpu.async_copy(msgb_hbm.at[idx_v], rows_v, sem).wait()
            for a in range(CA):
                for v in range(NV):
                    sl = pl.ds(v * L, L)
                    x = rows_v[a * MAX_NB, sl]
                    s = x
                    mx = x
                    for r in range(1, MAX_NB):
                        x = rows_v[a * MAX_NB + r, sl]
                        s = s + x
                        mx = jnp.maximum(mx, x)
                    agg_v[a, sl] = s * mx
            pltpu.sync_copy(agg_v, agg_hbm.at[pl.ds(pl.multiple_of(a0, CA), CA)])
            return carry

        lax.fori_loop(0, APW // CA, body, 0)

    return k(msg_bond, a2b_flat)


def gather_sub(a2t, msg_bond, b2a, b2revb):
    @functools.partial(
        pl.kernel,
        mesh=plsc.VectorSubcoreMesh(**_SC_MESH),
        out_type=jax.ShapeDtypeStruct((N_BONDS, H), jnp.float32),
        scratch_types=[
            pltpu.VMEM((CB,), jnp.int32),
            pltpu.VMEM((CB,), jnp.int32),
            pltpu.VMEM((CB, H), jnp.float32),
            pltpu.VMEM((CB, H), jnp.float32),
            pltpu.SemaphoreType.DMA,
        ],
    )
    def k(a2_hbm, msgb_hbm, b2a_hbm, b2revb_hbm, pre_hbm,
          idxa_v, idxr_v, bufa_v, bufr_v, sem):
        wid = lax.axis_index("s") * NC + lax.axis_index("c")

        def body(i, carry):
            b0 = pl.multiple_of(wid * BPW + i * CB, CB)
            pltpu.sync_copy(b2a_hbm.at[pl.ds(b0, CB)], idxa_v)
            pltpu.sync_copy(b2revb_hbm.at[pl.ds(b0, CB)], idxr_v)
            pltpu.async_copy(a2_hbm.at[idxa_v], bufa_v, sem).wait()
            pltpu.async_copy(msgb_hbm.at[idxr_v], bufr_v, sem).wait()
            for r in range(CB):
                for v in range(NV):
                    sl = pl.ds(v * L, L)
                    bufa_v[r, sl] = bufa_v[r, sl] - bufr_v[r, sl]
            pltpu.sync_copy(bufa_v, pre_hbm.at[pl.ds(b0, CB)])
            return carry

        lax.fori_loop(0, BPW // CB, body, 0)

    return k(a2t, msg_bond, b2a, b2revb)


def _pad2(x, r, c):
    return jnp.pad(x, ((0, r - x.shape[0]), (0, c - x.shape[1])))


def kernel(f_atoms, f_bonds, a2b, b2a, b2revb,
           W_i_atom, W_i_bond, W_h_0, W_h_1, W_lr):
    f_atoms_p = _pad2(f_atoms, NA, ATOM_FDIM)
    wia = _pad2(W_i_atom, ATOM_FDIM, H)
    wib = _pad2(W_i_bond, BOND_FDIM, H)
    wh0 = _pad2(W_h_0, H, H)
    wh1 = _pad2(W_h_1, H, H)
    wl_a = _pad2(W_lr[0:HID], H, H)
    wl_m = _pad2(W_lr[HID:2 * HID], H, H)
    wl_i = _pad2(W_lr[2 * HID:3 * HID], H, H)

    a2b_flat = jnp.pad(a2b.astype(jnp.int32), ((0, NA - N_ATOMS), (0, 0)))
    a2b_flat = a2b_flat.reshape(-1)
    b2a32 = b2a.astype(jnp.int32)
    b2revb32 = b2revb.astype(jnp.int32)

    input_atom = mm_relu(f_atoms_p, wia, bm=1024)
    input_bond = mm_relu(f_bonds, wib, bm=2000)

    msg_atom = input_atom
    msg_bond = input_bond
    for wh in (wh0, wh1):
        agg = gather_reduce(msg_bond, a2b_flat)
        msg_atom = upd_add(msg_atom, agg, bm=1024)
        pre = gather_sub(msg_atom, msg_bond, b2a32, b2revb32)
        msg_bond = comb_mm(pre, wh, input_bond, bm=2000)

    agg2 = gather_reduce(msg_bond, a2b_flat)
    out = final_mm(agg2, msg_atom, input_atom, wl_a, wl_m, wl_i, bm=1024)
    return out[1:N_ATOMS, 0:HID]


# speedup vs baseline: 1.4342x; 1.4342x over previous
import functools

import jax
import jax.numpy as jnp
from jax import lax
from jax.experimental import pallas as pl
from jax.experimental.pallas import tpu as pltpu
from jax.experimental.pallas import tpu_sc as plsc

N_ATOMS = 10000
MAX_NB = 32
N_BONDS = 320000
ATOM_FDIM = 128
BOND_FDIM = 144
HID = 300

L = 16
H = 384
NV = H // L
NC, NS = 2, 16
NW = NC * NS
NA = 10240
APW = NA // NW
BPW = N_BONDS // NW
CA = 2
CB = 80

_SC_MESH = dict(core_axis_name="c", subcore_axis_name="s")


def _mm_relu_body(x_ref, w_ref, o_ref):
    o_ref[...] = jnp.maximum(
        jnp.dot(x_ref[...], w_ref[...], preferred_element_type=jnp.float32), 0.0)


def mm_relu(x, w, bm):
    m, k = x.shape
    n = w.shape[1]
    return pl.pallas_call(
        _mm_relu_body,
        grid=(m // bm,),
        in_specs=[pl.BlockSpec((bm, k), lambda i: (i, 0)),
                  pl.BlockSpec((k, n), lambda i: (0, 0))],
        out_specs=pl.BlockSpec((bm, n), lambda i: (i, 0)),
        out_shape=jax.ShapeDtypeStruct((m, n), jnp.float32),
    )(x, w)


def _upd_body(ma_ref, agg_ref, man_ref):
    man_ref[...] = ma_ref[...] + agg_ref[...]


def upd_add(msg_atom, agg, bm):
    m = msg_atom.shape[0]
    return pl.pallas_call(
        _upd_body,
        grid=(m // bm,),
        in_specs=[pl.BlockSpec((bm, H), lambda i: (i, 0)),
                  pl.BlockSpec((bm, H), lambda i: (i, 0))],
        out_specs=pl.BlockSpec((bm, H), lambda i: (i, 0)),
        out_shape=jax.ShapeDtypeStruct((m, H), jnp.float32),
    )(msg_atom, agg)


def _comb_body(pre_ref, w_ref, ib_ref, o_ref):
    o_ref[...] = jnp.maximum(
        ib_ref[...]
        + jnp.dot(pre_ref[...], w_ref[...], preferred_element_type=jnp.float32),
        0.0)


def comb_mm(pre, w, input_bond, bm):
    m = pre.shape[0]
    return pl.pallas_call(
        _comb_body,
        grid=(m // bm,),
        in_specs=[pl.BlockSpec((bm, H), lambda i: (i, 0)),
                  pl.BlockSpec((H, H), lambda i: (0, 0)),
                  pl.BlockSpec((bm, H), lambda i: (i, 0))],
        out_specs=pl.BlockSpec((bm, H), lambda i: (i, 0)),
        out_shape=jax.ShapeDtypeStruct((m, H), jnp.float32),
    )(pre, w, input_bond)


def _final_body(xa_ref, xb_ref, xc_ref, wa_ref, wb_ref, wc_ref, o_ref):
    acc = jnp.dot(xa_ref[...], wa_ref[...], preferred_element_type=jnp.float32)
    acc += jnp.dot(xb_ref[...], wb_ref[...], preferred_element_type=jnp.float32)
    acc += jnp.dot(xc_ref[...], wc_ref[...], preferred_element_type=jnp.float32)
    o_ref[...] = jnp.maximum(acc, 0.0)


def final_mm(xa, xb, xc, wa, wb, wc, bm):
    m = xa.shape[0]
    xspec = pl.BlockSpec((bm, H), lambda i: (i, 0))
    wspec = pl.BlockSpec((H, H), lambda i: (0, 0))
    return pl.pallas_call(
        _final_body,
        grid=(m // bm,),
        in_specs=[xspec, xspec, xspec, wspec, wspec, wspec],
        out_specs=pl.BlockSpec((bm, H), lambda i: (i, 0)),
        out_shape=jax.ShapeDtypeStruct((m, H), jnp.float32),
    )(xa, xb, xc, wa, wb, wc)


def gather_reduce(msg_bond, a2b_flat):
    NCH = APW // CA
    NP = NCH // 2
    CROWS = CA * MAX_NB

    @functools.partial(
        pl.kernel,
        mesh=plsc.VectorSubcoreMesh(**_SC_MESH),
        out_type=jax.ShapeDtypeStruct((NA, H), jnp.float32),
        scratch_types=[
            pltpu.VMEM((CROWS,), jnp.int32),
            pltpu.VMEM((CROWS,), jnp.int32),
            pltpu.VMEM((CROWS, H), jnp.float32),
            pltpu.VMEM((CROWS, H), jnp.float32),
            pltpu.VMEM((CA, H), jnp.float32),
            pltpu.VMEM((CA, H), jnp.float32),
            pltpu.SemaphoreType.DMA,
            pltpu.SemaphoreType.DMA,
            pltpu.SemaphoreType.DMA,
            pltpu.SemaphoreType.DMA,
            pltpu.SemaphoreType.DMA,
            pltpu.SemaphoreType.DMA,
        ],
    )
    def k(msgb_hbm, a2b_hbm, agg_hbm, idx0, idx1, rows0, rows1, agg0, agg1,
          g0, g1, o0, o1, i0, i1):
        wid = lax.axis_index("s") * NC + lax.axis_index("c")
        idx = (idx0, idx1)
        rows = (rows0, rows1)
        agg = (agg0, agg1)
        gsem = (g0, g1)
        osem = (o0, o1)
        isem = (i0, i1)
        base_a = wid * APW

        def idesc(j, s):
            off = pl.multiple_of((base_a + j * CA) * MAX_NB, CROWS)
            return pltpu.make_async_copy(a2b_hbm.at[pl.ds(off, CROWS)],
                                         idx[s], isem[s])

        def gdesc(s):
            return pltpu.make_async_copy(msgb_hbm.at[idx[s]], rows[s],
                                         gsem[s])

        def out_sync(j, s):
            off = pl.multiple_of(base_a + j * CA, CA)
            pltpu.sync_copy(agg[s], agg_hbm.at[pl.ds(off, CA)])

        def compute(s):
            rv = rows[s]
            av = agg[s]

            def abody(a, c2):
                r0 = a * MAX_NB
                for v in range(NV):
                    sl = pl.ds(v * L, L)
                    x = rv[r0, sl]
                    sm = x
                    mx = x
                    for r in range(1, MAX_NB):
                        x = rv[r0 + r, sl]
                        sm = sm + x
                        mx = jnp.maximum(mx, x)
                    av[a, sl] = sm * mx
                return c2

            lax.fori_loop(0, CA, abody, 0)

        idesc(0, 0).start()
        idesc(1, 1).start()
        idesc(0, 0).wait()
        gdesc(0).start()
        idesc(1, 1).wait()
        gdesc(1).start()

        def body(p, carry):
            j0 = 2 * p
            j1 = j0 + 1
            gdesc(0).wait()
            idesc(j0 + 2, 0).start()
            compute(0)
            out_sync(j0, 0)
            idesc(j0 + 2, 0).wait()
            gdesc(0).start()
            gdesc(1).wait()
            idesc(j1 + 2, 1).start()
            compute(1)
            out_sync(j1, 1)
            idesc(j1 + 2, 1).wait()
            gdesc(1).start()
            return carry

        lax.fori_loop(0, NP - 1, body, 0)
        j0 = NCH - 2
        j1 = NCH - 1
        gdesc(0).wait()
        compute(0)
        out_sync(j0, 0)
        gdesc(1).wait()
        compute(1)
        out_sync(j1, 1)

    return k(msg_bond, a2b_flat)


def gather_sub(a2t, msg_bond, b2a, b2revb):
    NCH = BPW // CB
    NP = NCH // 2

    @functools.partial(
        pl.kernel,
        mesh=plsc.VectorSubcoreMesh(**_SC_MESH),
        out_type=jax.ShapeDtypeStruct((N_BONDS, H), jnp.float32),
        scratch_types=[
            pltpu.VMEM((CB,), jnp.int32),
            pltpu.VMEM((CB,), jnp.int32),
            pltpu.VMEM((CB,), jnp.int32),
            pltpu.VMEM((CB,), jnp.int32),
            pltpu.VMEM((CB, H), jnp.float32),
            pltpu.VMEM((CB, H), jnp.float32),
            pltpu.VMEM((CB, H), jnp.float32),
            pltpu.VMEM((CB, H), jnp.float32),
            pltpu.SemaphoreType.DMA,
            pltpu.SemaphoreType.DMA,
            pltpu.SemaphoreType.DMA,
            pltpu.SemaphoreType.DMA,
            pltpu.SemaphoreType.DMA,
            pltpu.SemaphoreType.DMA,
        ],
    )
    def k(a2_hbm, msgb_hbm, b2a_hbm, b2revb_hbm, pre_hbm,
          idxa0, idxa1, idxr0, idxr1, bufa0, bufa1, bufr0, bufr1,
          g0, g1, o0, o1, i0, i1):
        wid = lax.axis_index("s") * NC + lax.axis_index("c")
        idxa = (idxa0, idxa1)
        idxr = (idxr0, idxr1)
        bufa = (bufa0, bufa1)
        bufr = (bufr0, bufr1)
        gsem = (g0, g1)
        osem = (o0, o1)
        isem = (i0, i1)
        base_b = pl.multiple_of(wid * BPW, CB)

        def idesc(j, s):
            off = pl.multiple_of(base_b + j * CB, CB)
            return (pltpu.make_async_copy(b2a_hbm.at[pl.ds(off, CB)],
                                          idxa[s], isem[s]),
                    pltpu.make_async_copy(b2revb_hbm.at[pl.ds(off, CB)],
                                          idxr[s], isem[s]))

        def idx_start(j, s):
            da, dr = idesc(j, s)
            da.start()
            dr.start()

        def idx_wait(j, s):
            da, dr = idesc(j, s)
            da.wait()
            dr.wait()

        def gdescs(s):
            return (pltpu.make_async_copy(a2_hbm.at[idxa[s]], bufa[s],
                                          gsem[s]),
                    pltpu.make_async_copy(msgb_hbm.at[idxr[s]], bufr[s],
                                          gsem[s]))

        def start_g(s):
            da, dr = gdescs(s)
            da.start()
            dr.start()

        def wait_g(s):
            da, dr = gdescs(s)
            da.wait()
            dr.wait()

        def odesc(j, s):
            off = pl.multiple_of(base_b + j * CB, CB)
            return pltpu.make_async_copy(bufa[s], pre_hbm.at[pl.ds(off, CB)],
                                         osem[s])

        def compute(s):
            av = bufa[s]
            rv = bufr[s]

            def rbody(r, c2):
                for v in range(NV):
                    sl = pl.ds(v * L, L)
                    av[r, sl] = av[r, sl] - rv[r, sl]
                return c2

            lax.fori_loop(0, CB, rbody, 0)

        def idx_sync(j, s):
            idx_start(j, s)
            idx_wait(j, s)

        def out_sync(j, s):
            d = odesc(j, s)
            d.start()
            d.wait()

        def body(i, carry):
            b0 = pl.multiple_of(base_b + i * CB, CB)
            pltpu.sync_copy(b2a_hbm.at[pl.ds(b0, CB)], idxa0)
            pltpu.sync_copy(b2revb_hbm.at[pl.ds(b0, CB)], idxr0)
            pltpu.async_copy(a2_hbm.at[idxa0], bufa0, g0).wait()
            pltpu.async_copy(msgb_hbm.at[idxr0], bufr0, g0).wait()
            compute(0)
            pltpu.sync_copy(bufa0, pre_hbm.at[pl.ds(b0, CB)])
            return carry

        lax.fori_loop(0, NCH, body, 0)

    return k(a2t, msg_bond, b2a, b2revb)


def _pad2(x, r, c):
    return jnp.pad(x, ((0, r - x.shape[0]), (0, c - x.shape[1])))


def kernel(f_atoms, f_bonds, a2b, b2a, b2revb,
           W_i_atom, W_i_bond, W_h_0, W_h_1, W_lr):
    f_atoms_p = _pad2(f_atoms, NA, ATOM_FDIM)
    wia = _pad2(W_i_atom, ATOM_FDIM, H)
    wib = _pad2(W_i_bond, BOND_FDIM, H)
    wh0 = _pad2(W_h_0, H, H)
    wh1 = _pad2(W_h_1, H, H)
    wl_a = _pad2(W_lr[0:HID], H, H)
    wl_m = _pad2(W_lr[HID:2 * HID], H, H)
    wl_i = _pad2(W_lr[2 * HID:3 * HID], H, H)

    a2b_flat = jnp.pad(a2b.astype(jnp.int32), ((0, NA - N_ATOMS), (0, 0)))
    a2b_flat = a2b_flat.reshape(-1)
    b2a32 = b2a.astype(jnp.int32)
    b2revb32 = b2revb.astype(jnp.int32)

    input_atom = mm_relu(f_atoms_p, wia, bm=1024)
    input_bond = mm_relu(f_bonds, wib, bm=2000)

    msg_atom = input_atom
    msg_bond = input_bond
    for wh in (wh0, wh1):
        agg = gather_reduce(msg_bond, a2b_flat)
        msg_atom = upd_add(msg_atom, agg, bm=1024)
        pre = gather_sub(msg_atom, msg_bond, b2a32, b2revb32)
        msg_bond = comb_mm(pre, wh, input_bond, bm=2000)

    agg2 = gather_reduce(msg_bond, a2b_flat)
    out = final_mm(agg2, msg_atom, input_atom, wl_a, wl_m, wl_i, bm=1024)
    return out[1:N_ATOMS, 0:HID]
